# Initial kernel scaffold; baseline (speedup 1.0000x reference)
import jax
import jax.numpy as jnp
from jax.experimental import pallas as pl


def kernel(x, edge_index, W1l, W1r, b1, W2l, W2r, b2):
    raise NotImplementedError("write your pallas kernel here")


# speedup vs baseline: 5.7053x; 5.7053x over previous
import jax
import jax.numpy as jnp
from jax import lax
from jax.experimental import pallas as pl
from jax.experimental.pallas import tpu as pltpu
from jax.experimental.pallas import tpu_sc as plsc

CHUNK = 128
LANES = 16


def _sc_geometry(n, e):
  info = plsc.get_sparse_core_info()
  nc, ns = info.num_cores, info.num_subcores
  nw = nc * ns
  assert e % CHUNK == 0 and n % 8 == 0
  base_cnt, rem = divmod(e // CHUNK, nw)
  rows_per_tile = 8 * ((n // 8 + ns - 1) // ns)
  assert n >= rows_per_tile
  return nc, ns, nw, base_cnt, rem, rows_per_tile


def _fill(ref, nrows, ncols, val):
  per_row = ncols // LANES

  def st(i, _):
    r = i // per_row
    c = (i % per_row) * LANES
    ref[r, pl.ds(c, LANES)] = jnp.full((LANES,), val, jnp.float32)
    return 0

  lax.fori_loop(0, nrows * per_row, st, 0)


def _zero_acc(rows, acc, row0, rows_per_tile):
  done = 0
  for _ in range((rows_per_tile + CHUNK - 1) // CHUNK):
    cnt = min(CHUNK, rows_per_tile - done)
    pltpu.sync_copy(rows.at[pl.ds(0, cnt)], acc.at[pl.ds(row0 + done, cnt)])
    done += cnt


def _seg_sum_sc(n, d, e):
  nc, ns, nw, base_cnt, rem, rows_per_tile = _sc_geometry(n, e)
  assert d % LANES == 0
  mesh = plsc.VectorSubcoreMesh(core_axis_name="c", subcore_axis_name="s")

  def body(table, src, dst, out, acc, src_idx, dst_idx, rows, gsem, ssem):
    cid = lax.axis_index("c")
    sid = lax.axis_index("s")
    wid = sid * nc + cid

    _fill(rows, CHUNK, d, 0.0)
    row0 = jnp.minimum(sid * rows_per_tile, n - rows_per_tile)
    _zero_acc(rows, acc, row0, rows_per_tile)
    plsc.subcore_barrier()

    start = wid * base_cnt + jnp.minimum(wid, rem)
    count = base_cnt + jnp.where(wid < rem, 1, 0)

    def chunk_body(i, _):
      off = (start + i) * CHUNK
      pltpu.sync_copy(src.at[pl.ds(off, CHUNK)], src_idx)
      pltpu.sync_copy(dst.at[pl.ds(off, CHUNK)], dst_idx)
      pltpu.async_copy(table.at[src_idx], rows, gsem).wait()
      pltpu.async_copy(rows, acc.at[dst_idx], ssem, add=True).wait()
      return 0

    lax.fori_loop(0, count, chunk_body, 0)
    plsc.subcore_barrier()

    pltpu.sync_copy(acc.at[pl.ds(row0, rows_per_tile)],
                    out.at[cid, pl.ds(row0, rows_per_tile)])

  return pl.kernel(
      body,
      out_type=[jax.ShapeDtypeStruct((nc, n, d), jnp.float32)],
      mesh=mesh,
      scratch_types=[
          pltpu.VMEM_SHARED((n, d), jnp.float32),
          pltpu.VMEM((CHUNK,), jnp.int32),
          pltpu.VMEM((CHUNK,), jnp.int32),
          pltpu.VMEM((CHUNK, d), jnp.float32),
          pltpu.SemaphoreType.DMA,
          pltpu.SemaphoreType.DMA,
      ],
  )


def _deg_hist_sc(n, d, e):
  nc, ns, nw, base_cnt, rem, rows_per_tile = _sc_geometry(n, e)
  mesh = plsc.VectorSubcoreMesh(core_axis_name="c", subcore_axis_name="s")

  def body(dst, out, acc, dst_idx, rows, ssem):
    cid = lax.axis_index("c")
    sid = lax.axis_index("s")
    wid = sid * nc + cid

    _fill(rows, CHUNK, d, 0.0)
    row0 = jnp.minimum(sid * rows_per_tile, n - rows_per_tile)
    _zero_acc(rows, acc, row0, rows_per_tile)
    _fill(rows, CHUNK, d, 1.0)
    plsc.subcore_barrier()

    start = wid * base_cnt + jnp.minimum(wid, rem)
    count = base_cnt + jnp.where(wid < rem, 1, 0)

    def chunk_body(i, _):
      off = (start + i) * CHUNK
      pltpu.sync_copy(dst.at[pl.ds(off, CHUNK)], dst_idx)
      pltpu.async_copy(rows, acc.at[dst_idx], ssem, add=True).wait()
      return 0

    lax.fori_loop(0, count, chunk_body, 0)
    plsc.subcore_barrier()

    pltpu.sync_copy(acc.at[pl.ds(row0, rows_per_tile)],
                    out.at[cid, pl.ds(row0, rows_per_tile)])

  return pl.kernel(
      body,
      out_type=[jax.ShapeDtypeStruct((nc, n, d), jnp.float32)],
      mesh=mesh,
      scratch_types=[
          pltpu.VMEM_SHARED((n, d), jnp.float32),
          pltpu.VMEM((CHUNK,), jnp.int32),
          pltpu.VMEM((CHUNK, d), jnp.float32),
          pltpu.SemaphoreType.DMA,
      ],
  )


def _tc_pre(x, Wl, Wr, b, br):
  n, d = x.shape

  def body(x_ref, wl_ref, wr_ref, b_ref, z_ref, y_ref):
    xb = x_ref[...]
    z_ref[...] = jnp.dot(xb, wl_ref[...],
                         preferred_element_type=jnp.float32) + b_ref[...]
    y_ref[...] = jnp.dot(xb, wr_ref[...], preferred_element_type=jnp.float32)

  return pl.pallas_call(
      body,
      grid=(n // br,),
      in_specs=[
          pl.BlockSpec((br, d), lambda i: (i, 0)),
          pl.BlockSpec((d, d), lambda i: (0, 0)),
          pl.BlockSpec((d, d), lambda i: (0, 0)),
          pl.BlockSpec((1, d), lambda i: (0, 0)),
      ],
      out_specs=[
          pl.BlockSpec((br, d), lambda i: (i, 0)),
          pl.BlockSpec((br, d), lambda i: (i, 0)),
      ],
      out_shape=[jax.ShapeDtypeStruct((n, d), jnp.float32)] * 2,
  )(x, Wl, Wr, b.reshape(1, d))


def _tc_mid(z1, p1, pdeg, Wl, Wr, b, br):
  n, d = z1.shape

  def body(z1_ref, p_ref, pd_ref, wl_ref, wr_ref, b_ref, z2_ref, y2_ref):
    s = p_ref[0] + p_ref[1]
    deg = pd_ref[0, :, 0:1] + pd_ref[1, :, 0:1]
    inv = 1.0 / jnp.maximum(deg, 1.0)
    h = jnp.maximum(z1_ref[...] + s * inv, 0.0)
    z2_ref[...] = jnp.dot(h, wl_ref[...],
                          preferred_element_type=jnp.float32) + b_ref[...]
    y2_ref[...] = jnp.dot(h, wr_ref[...], preferred_element_type=jnp.float32)

  return pl.pallas_call(
      body,
      grid=(n // br,),
      in_specs=[
          pl.BlockSpec((br, d), lambda i: (i, 0)),
          pl.BlockSpec((2, br, d), lambda i: (0, i, 0)),
          pl.BlockSpec((2, br, d), lambda i: (0, i, 0)),
          pl.BlockSpec((d, d), lambda i: (0, 0)),
          pl.BlockSpec((d, d), lambda i: (0, 0)),
          pl.BlockSpec((1, d), lambda i: (0, 0)),
      ],
      out_specs=[
          pl.BlockSpec((br, d), lambda i: (i, 0)),
          pl.BlockSpec((br, d), lambda i: (i, 0)),
      ],
      out_shape=[jax.ShapeDtypeStruct((n, d), jnp.float32)] * 2,
  )(z1, p1, pdeg, Wl, Wr, b.reshape(1, d))


def _tc_post(z2, p2, pdeg, br):
  n, d = z2.shape

  def body(z2_ref, p_ref, pd_ref, out_ref):
    s = p_ref[0] + p_ref[1]
    deg = pd_ref[0, :, 0:1] + pd_ref[1, :, 0:1]
    out_ref[...] = z2_ref[...] + s * (1.0 / jnp.maximum(deg, 1.0))

  return pl.pallas_call(
      body,
      grid=(n // br,),
      in_specs=[
          pl.BlockSpec((br, d), lambda i: (i, 0)),
          pl.BlockSpec((2, br, d), lambda i: (0, i, 0)),
          pl.BlockSpec((2, br, d), lambda i: (0, i, 0)),
      ],
      out_specs=pl.BlockSpec((br, d), lambda i: (i, 0)),
      out_shape=jax.ShapeDtypeStruct((n, d), jnp.float32),
  )(z2, p2, pdeg)


def kernel(x, edge_index, W1l, W1r, b1, W2l, W2r, b2):
  n, d = x.shape
  e = edge_index.shape[1]
  br = 1000 if n % 1000 == 0 else 8
  src = edge_index[0]
  dst = edge_index[1]

  seg = _seg_sum_sc(n, d, e)
  pdeg, = _deg_hist_sc(n, d, e)(dst)
  z1, y1 = _tc_pre(x, W1l, W1r, b1, br)
  p1, = seg(y1, src, dst)
  z2, y2 = _tc_mid(z1, p1, pdeg, W2l, W2r, b2, br)
  p2, = seg(y2, src, dst)
  return _tc_post(z2, p2, pdeg, br)
